# baseline (speedup 1.0000x reference)
import jax
import jax.numpy as jnp
import numpy as np
from jax.experimental import pallas as pl
from jax.experimental.pallas import tpu as pltpu

B, T = 16, 4096
D_DEMO = 16
N_MOD = 37
N_POS = 10
NPH = N_POS // 2
PHI_W, LATENT = 128, 128
DOT, HEADS = 64, 4
D_IN = N_POS + 1 + N_MOD
TP = T + 8
AUG = 2 * PHI_W


def _main_body(lengths_ref, times_ref, values_ref, meas_ref, demo_ref,
               scw_ref,
               wd1_ref, bd1_ref, wd2_ref, bd2_ref,
               wpv_ref, wm_ref, waug_ref, baug_ref,
               w2_ref, b2_ref, w3_ref, b3_ref, w4_ref, b4_ref,
               out_ref, h1_ref):
    b = pl.program_id(0)
    L = lengths_ref[b]

    tr = times_ref[...].reshape(1, T)
    tv = jnp.concatenate([tr, jnp.full((1, T), 1.0, jnp.float32)], axis=0)
    sc = jax.lax.dot_general(scw_ref[...], tv, (((1,), (0,)), ((), ())),
                             preferred_element_type=jnp.float32)
    sub = jax.lax.broadcasted_iota(jnp.int32, (N_POS + 1, 1), 0)
    posv = jnp.where(sub < N_POS, jnp.sin(sc), values_ref[...].reshape(1, T))

    u = (jax.lax.dot_general(posv, wpv_ref[...], (((0,), (0,)), ((), ())),
                             preferred_element_type=jnp.float32)
         + jnp.dot(meas_ref[0], wm_ref[...], preferred_element_type=jnp.float32)
         + baug_ref[...])
    h1_data = jnp.maximum(u[:, :PHI_W], 0.0)
    pre_data = u[:, PHI_W:PHI_W + HEADS]

    dh = jnp.maximum(demo_ref[0] @ wd1_ref[...] + bd1_ref[...], 0.0)
    de = dh @ wd2_ref[...] + bd2_ref[...]
    ud = jnp.dot(de, waug_ref[...], preferred_element_type=jnp.float32) + baug_ref[...]
    h1_demo = jnp.maximum(ud[:, :PHI_W], 0.0)
    pre_demo = ud[:, PHI_W:PHI_W + HEADS]

    h1_ref[pl.ds(T, 8), :] = jnp.zeros((8, PHI_W), jnp.bfloat16)
    h1_ref[pl.ds(0, T), :] = h1_data.astype(jnp.bfloat16)
    h1_ref[pl.ds(T, 1), :] = h1_demo.astype(jnp.bfloat16)

    h = jnp.maximum(jnp.dot(h1_ref[...], w2_ref[...], preferred_element_type=jnp.float32) + b2_ref[...], 0.0)
    h = jnp.maximum(jnp.dot(h.astype(jnp.bfloat16), w3_ref[...], preferred_element_type=jnp.float32) + b3_ref[...], 0.0)
    enc = jnp.maximum(jnp.dot(h.astype(jnp.bfloat16), w4_ref[...], preferred_element_type=jnp.float32) + b4_ref[...], 0.0)

    pre_t = jnp.concatenate(
        [pre_data.T, pre_demo.reshape(HEADS, 1),
         jnp.zeros((HEADS, 7), jnp.float32)], axis=1)
    lane = jax.lax.broadcasted_iota(jnp.int32, (1, TP), 1)
    valid = (lane < L) | (lane == T)
    prem = jnp.where(valid, pre_t, -jnp.inf)
    mx = jnp.max(prem, axis=1, keepdims=True)
    e = jnp.exp(prem - mx)
    s = jnp.sum(e, axis=1, keepdims=True)
    attn = (e / s).astype(jnp.bfloat16)

    out = jax.lax.dot_general(attn, enc.astype(jnp.bfloat16),
                              (((1,), (0,)), ((), ())),
                              preferred_element_type=jnp.float32)
    out_ref[...] = out.reshape(1, 1, HEADS * LATENT)


def _rho_body(x_ref, w1_ref, b1_ref, w2_ref, b2_ref, w3_ref, b3_ref,
              w4_ref, b4_ref, out_ref):
    h = jnp.maximum(jnp.dot(x_ref[...], w1_ref[...], preferred_element_type=jnp.float32) + b1_ref[...], 0.0)
    h = jnp.maximum(jnp.dot(h, w2_ref[...], preferred_element_type=jnp.float32) + b2_ref[...], 0.0)
    h = jnp.maximum(jnp.dot(h, w3_ref[...], preferred_element_type=jnp.float32) + b3_ref[...], 0.0)
    y = jnp.dot(h, w4_ref[...], preferred_element_type=jnp.float32) + b4_ref[...]
    out_ref[...] = jax.nn.sigmoid(y)


@jax.jit
def kernel(demo, times, values, measurements, lengths, timescales,
           demo_w, phi_w, psi_w, W_k, W_q, rho_w):
    del psi_w

    a_proj = jnp.einsum('khd,hd->kh', W_k.reshape(-1, HEADS, DOT), W_q)[:D_IN]
    a_proj = a_proj / np.sqrt(float(DOT))
    (w1, b1) = phi_w[0]
    waug = jnp.zeros((D_IN, AUG), jnp.float32)
    waug = waug.at[:, :PHI_W].set(w1).at[:, PHI_W:PHI_W + HEADS].set(a_proj)
    baug = jnp.zeros((1, AUG), jnp.float32).at[:, :PHI_W].set(b1)
    wpv = waug[:N_POS + 1]
    wm = waug[N_POS + 1:]
    recip = 1.0 / timescales
    inv11 = jnp.concatenate([recip, recip, jnp.ones((1,), jnp.float32)])
    off11 = jnp.concatenate([jnp.zeros((NPH,), jnp.float32),
                             jnp.full((NPH,), np.pi / 2, jnp.float32),
                             jnp.zeros((1,), jnp.float32)])
    scw = jnp.stack([inv11, off11], axis=1)

    t2 = times.reshape(B, 1, T)
    v2 = values.reshape(B, 1, T)
    d2 = demo.reshape(B, 1, D_DEMO)
    (wd1, bd1), (wd2, bd2) = demo_w
    flat_phi234 = []
    for w, bb in phi_w[1:]:
        flat_phi234 += [w.astype(jnp.bfloat16), bb.reshape(1, -1)]

    rep = lambda s: pl.BlockSpec(s, lambda b, L: tuple(0 for _ in s))
    w_specs = [rep((N_POS + 1, 2)),
               rep(wd1.shape), rep((1, PHI_W)), rep(wd2.shape), rep((1, D_IN)),
               rep(wpv.shape), rep(wm.shape), rep(waug.shape),
               rep((1, AUG))]
    for w, bb in phi_w[1:]:
        w_specs += [rep(w.shape), rep((1, w.shape[1]))]

    grid_spec = pltpu.PrefetchScalarGridSpec(
        num_scalar_prefetch=1,
        grid=(B,),
        in_specs=[
            pl.BlockSpec((1, 1, T), lambda b, L: (b, 0, 0)),
            pl.BlockSpec((1, 1, T), lambda b, L: (b, 0, 0)),
            pl.BlockSpec((1, T, N_MOD), lambda b, L: (b, 0, 0)),
            pl.BlockSpec((1, 1, D_DEMO), lambda b, L: (b, 0, 0)),
        ] + w_specs,
        out_specs=pl.BlockSpec((1, 1, HEADS * LATENT), lambda b, L: (b, 0, 0)),
        scratch_shapes=[pltpu.VMEM((TP, PHI_W), jnp.bfloat16)],
    )

    aggregated = pl.pallas_call(
        _main_body,
        grid_spec=grid_spec,
        out_shape=jax.ShapeDtypeStruct((B, 1, HEADS * LATENT), jnp.float32),
        compiler_params=pltpu.CompilerParams(
            dimension_semantics=("arbitrary",)),
    )(lengths, t2, v2, measurements, d2,
      scw, wd1, bd1.reshape(1, -1), wd2, bd2.reshape(1, -1),
      wpv, wm, waug, baug, *flat_phi234)
    aggregated = aggregated.reshape(B, HEADS * LATENT)

    flat_rho = []
    for w, bb in rho_w:
        flat_rho += [w, bb.reshape(1, -1)]
    out = pl.pallas_call(
        _rho_body,
        out_shape=jax.ShapeDtypeStruct((B, 1), jnp.float32),
    )(aggregated, *flat_rho)
    return out


# speedup vs baseline: 1.1911x; 1.1911x over previous
import jax
import jax.numpy as jnp
import numpy as np
from jax.experimental import pallas as pl
from jax.experimental.pallas import tpu as pltpu

B, T = 16, 4096
D_DEMO = 16
N_MOD = 37
N_POS = 10
NPH = N_POS // 2
PHI_W, LATENT = 128, 128
DOT, HEADS = 64, 4
D_IN = N_POS + 1 + N_MOD
TP = T + 8
AUG = 2 * PHI_W


def _main_body(lengths_ref, times_ref, values_ref, meas_ref, demo_ref,
               scw_ref,
               wd1_ref, bd1_ref, wd2_ref, bd2_ref,
               wpv_ref, wm_ref, waug_ref, baug_ref,
               w2_ref, b2_ref, w3_ref, b3_ref, w4_ref, b4_ref,
               out_ref, h1_ref):
    b = pl.program_id(0)
    L = lengths_ref[b]

    tr = times_ref[...].reshape(1, T)
    tv = jnp.concatenate([tr, jnp.full((1, T), 1.0, jnp.float32)], axis=0)
    sc = jax.lax.dot_general(scw_ref[...], tv, (((1,), (0,)), ((), ())),
                             preferred_element_type=jnp.float32)
    sub = jax.lax.broadcasted_iota(jnp.int32, (N_POS + 1, 1), 0)
    posv = jnp.where(sub < N_POS, jnp.sin(sc), values_ref[...].reshape(1, T))

    u = (jax.lax.dot_general(posv, wpv_ref[...], (((0,), (0,)), ((), ())),
                             preferred_element_type=jnp.float32)
         + jax.lax.dot_general(meas_ref[:, 0, 0, :], wm_ref[...],
                               (((0,), (0,)), ((), ())),
                               preferred_element_type=jnp.float32)
         + baug_ref[...])
    h1_data = jnp.maximum(u[:, :PHI_W], 0.0)
    pre_data = u[:, PHI_W:PHI_W + HEADS]

    dh = jnp.maximum(demo_ref[0] @ wd1_ref[...] + bd1_ref[...], 0.0)
    de = dh @ wd2_ref[...] + bd2_ref[...]
    ud = jnp.dot(de, waug_ref[...], preferred_element_type=jnp.float32) + baug_ref[...]
    h1_demo = jnp.maximum(ud[:, :PHI_W], 0.0)
    pre_demo = ud[:, PHI_W:PHI_W + HEADS]

    h1_ref[pl.ds(T, 8), :] = jnp.zeros((8, PHI_W), jnp.bfloat16)
    h1_ref[pl.ds(0, T), :] = h1_data.astype(jnp.bfloat16)
    h1_ref[pl.ds(T, 1), :] = h1_demo.astype(jnp.bfloat16)

    h = jnp.maximum(jnp.dot(h1_ref[...], w2_ref[...], preferred_element_type=jnp.float32) + b2_ref[...], 0.0)
    h = jnp.maximum(jnp.dot(h.astype(jnp.bfloat16), w3_ref[...], preferred_element_type=jnp.float32) + b3_ref[...], 0.0)
    enc = jnp.maximum(jnp.dot(h.astype(jnp.bfloat16), w4_ref[...], preferred_element_type=jnp.float32) + b4_ref[...], 0.0)

    pre_t = jnp.concatenate(
        [pre_data.T, pre_demo.reshape(HEADS, 1),
         jnp.zeros((HEADS, 7), jnp.float32)], axis=1)
    lane = jax.lax.broadcasted_iota(jnp.int32, (1, TP), 1)
    valid = (lane < L) | (lane == T)
    prem = jnp.where(valid, pre_t, -jnp.inf)
    mx = jnp.max(prem, axis=1, keepdims=True)
    e = jnp.exp(prem - mx)
    s = jnp.sum(e, axis=1, keepdims=True)
    attn = (e / s).astype(jnp.bfloat16)

    out = jax.lax.dot_general(attn, enc.astype(jnp.bfloat16),
                              (((1,), (0,)), ((), ())),
                              preferred_element_type=jnp.float32)
    out_ref[...] = out.reshape(1, 1, HEADS * LATENT)


def _rho_body(x_ref, w1_ref, b1_ref, w2_ref, b2_ref, w3_ref, b3_ref,
              w4_ref, b4_ref, out_ref):
    h = jnp.maximum(jnp.dot(x_ref[...], w1_ref[...], preferred_element_type=jnp.float32) + b1_ref[...], 0.0)
    h = jnp.maximum(jnp.dot(h, w2_ref[...], preferred_element_type=jnp.float32) + b2_ref[...], 0.0)
    h = jnp.maximum(jnp.dot(h, w3_ref[...], preferred_element_type=jnp.float32) + b3_ref[...], 0.0)
    y = jnp.dot(h, w4_ref[...], preferred_element_type=jnp.float32) + b4_ref[...]
    out_ref[...] = jax.nn.sigmoid(y)


@jax.jit
def kernel(demo, times, values, measurements, lengths, timescales,
           demo_w, phi_w, psi_w, W_k, W_q, rho_w):
    del psi_w

    a_proj = jnp.einsum('khd,hd->kh', W_k.reshape(-1, HEADS, DOT), W_q)[:D_IN]
    a_proj = a_proj / np.sqrt(float(DOT))
    (w1, b1) = phi_w[0]
    waug = jnp.zeros((D_IN, AUG), jnp.float32)
    waug = waug.at[:, :PHI_W].set(w1).at[:, PHI_W:PHI_W + HEADS].set(a_proj)
    baug = jnp.zeros((1, AUG), jnp.float32).at[:, :PHI_W].set(b1)
    wpv = waug[:N_POS + 1]
    wm = waug[N_POS + 1:]
    recip = 1.0 / timescales
    inv11 = jnp.concatenate([recip, recip, jnp.ones((1,), jnp.float32)])
    off11 = jnp.concatenate([jnp.zeros((NPH,), jnp.float32),
                             jnp.full((NPH,), np.pi / 2, jnp.float32),
                             jnp.zeros((1,), jnp.float32)])
    scw = jnp.stack([inv11, off11], axis=1)

    t2 = times.reshape(B, 1, T)
    v2 = values.reshape(B, 1, T)
    m2 = measurements.transpose(2, 0, 1).reshape(N_MOD, B, 1, T)
    d2 = demo.reshape(B, 1, D_DEMO)
    (wd1, bd1), (wd2, bd2) = demo_w
    flat_phi234 = []
    for w, bb in phi_w[1:]:
        flat_phi234 += [w.astype(jnp.bfloat16), bb.reshape(1, -1)]

    rep = lambda s: pl.BlockSpec(s, lambda b, L: tuple(0 for _ in s))
    w_specs = [rep((N_POS + 1, 2)),
               rep(wd1.shape), rep((1, PHI_W)), rep(wd2.shape), rep((1, D_IN)),
               rep(wpv.shape), rep(wm.shape), rep(waug.shape),
               rep((1, AUG))]
    for w, bb in phi_w[1:]:
        w_specs += [rep(w.shape), rep((1, w.shape[1]))]

    grid_spec = pltpu.PrefetchScalarGridSpec(
        num_scalar_prefetch=1,
        grid=(B,),
        in_specs=[
            pl.BlockSpec((1, 1, T), lambda b, L: (b, 0, 0)),
            pl.BlockSpec((1, 1, T), lambda b, L: (b, 0, 0)),
            pl.BlockSpec((N_MOD, 1, 1, T), lambda b, L: (0, b, 0, 0)),
            pl.BlockSpec((1, 1, D_DEMO), lambda b, L: (b, 0, 0)),
        ] + w_specs,
        out_specs=pl.BlockSpec((1, 1, HEADS * LATENT), lambda b, L: (b, 0, 0)),
        scratch_shapes=[pltpu.VMEM((TP, PHI_W), jnp.bfloat16)],
    )

    aggregated = pl.pallas_call(
        _main_body,
        grid_spec=grid_spec,
        out_shape=jax.ShapeDtypeStruct((B, 1, HEADS * LATENT), jnp.float32),
        compiler_params=pltpu.CompilerParams(
            dimension_semantics=("arbitrary",)),
    )(lengths, t2, v2, m2, d2,
      scw, wd1, bd1.reshape(1, -1), wd2, bd2.reshape(1, -1),
      wpv, wm, waug, baug, *flat_phi234)
    aggregated = aggregated.reshape(B, HEADS * LATENT)

    flat_rho = []
    for w, bb in rho_w:
        flat_rho += [w, bb.reshape(1, -1)]
    out = pl.pallas_call(
        _rho_body,
        out_shape=jax.ShapeDtypeStruct((B, 1), jnp.float32),
    )(aggregated, *flat_rho)
    return out


# speedup vs baseline: 1.1926x; 1.0013x over previous
import jax
import jax.numpy as jnp
import numpy as np
from jax.experimental import pallas as pl
from jax.experimental.pallas import tpu as pltpu

B, T = 16, 4096
D_DEMO = 16
N_MOD = 37
N_POS = 10
NPH = N_POS // 2
PHI_W, LATENT = 128, 128
DOT, HEADS = 64, 4
D_IN = N_POS + 1 + N_MOD
TP = T + 8
AUG = 2 * PHI_W


def _main_body(lengths_ref, times_ref, values_ref, meas_ref, demo_ref,
               scw_ref,
               wd1_ref, bd1_ref, wd2_ref, bd2_ref,
               wpv_ref, wm_ref, waug_ref, baug_ref,
               w2_ref, b2_ref, w3_ref, b3_ref, w4_ref, b4_ref,
               out_ref, h1_ref):
    b = pl.program_id(0)
    L = lengths_ref[b]

    tr = times_ref[...].reshape(1, T)
    tv = jnp.concatenate([tr, jnp.full((1, T), 1.0, jnp.float32)], axis=0)
    sc = jax.lax.dot_general(scw_ref[...], tv, (((1,), (0,)), ((), ())),
                             preferred_element_type=jnp.float32)
    sub = jax.lax.broadcasted_iota(jnp.int32, (N_POS + 1, 1), 0)
    posv = jnp.where(sub < N_POS, jnp.sin(sc), values_ref[...].reshape(1, T))

    u = (jax.lax.dot_general(posv, wpv_ref[...], (((0,), (0,)), ((), ())),
                             preferred_element_type=jnp.float32)
         + jax.lax.dot_general(meas_ref[:, 0, 0, :], wm_ref[...],
                               (((0,), (0,)), ((), ())),
                               preferred_element_type=jnp.float32)
         + baug_ref[...])
    h1_data = jnp.maximum(u[:, :PHI_W], 0.0)
    pre_data = u[:, PHI_W:PHI_W + HEADS]

    dh = jnp.maximum(demo_ref[0] @ wd1_ref[...] + bd1_ref[...], 0.0)
    de = dh @ wd2_ref[...] + bd2_ref[...]
    ud = jnp.dot(de, waug_ref[...], preferred_element_type=jnp.float32) + baug_ref[...]
    h1_demo = jnp.maximum(ud[:, :PHI_W], 0.0)
    pre_demo = ud[:, PHI_W:PHI_W + HEADS]

    h1_ref[pl.ds(T, 8), :] = jnp.zeros((8, PHI_W), jnp.bfloat16)
    h1_ref[pl.ds(0, T), :] = h1_data.astype(jnp.bfloat16)
    h1_ref[pl.ds(T, 1), :] = h1_demo.astype(jnp.bfloat16)

    h = jnp.maximum(jnp.dot(h1_ref[...], w2_ref[...], preferred_element_type=jnp.float32) + b2_ref[...], 0.0)
    h = jnp.maximum(jnp.dot(h.astype(jnp.bfloat16), w3_ref[...], preferred_element_type=jnp.float32) + b3_ref[...], 0.0)
    enc = jnp.maximum(jnp.dot(h.astype(jnp.bfloat16), w4_ref[...], preferred_element_type=jnp.float32) + b4_ref[...], 0.0)

    pre_t = jnp.concatenate(
        [pre_data.T, pre_demo.reshape(HEADS, 1),
         jnp.zeros((HEADS, 7), jnp.float32)], axis=1)
    lane = jax.lax.broadcasted_iota(jnp.int32, (1, TP), 1)
    valid = (lane < L) | (lane == T)
    prem = jnp.where(valid, pre_t, -jnp.inf)
    mx = jnp.max(prem, axis=1, keepdims=True)
    e = jnp.exp(prem - mx)
    s = jnp.sum(e, axis=1, keepdims=True)
    attn = (e / s).astype(jnp.bfloat16)

    out = jax.lax.dot_general(attn, enc.astype(jnp.bfloat16),
                              (((1,), (0,)), ((), ())),
                              preferred_element_type=jnp.float32)
    out_ref[...] = out.reshape(1, 1, HEADS * LATENT)


def _rho_body(x_ref, w1_ref, b1_ref, w2_ref, b2_ref, w3_ref, b3_ref,
              w4_ref, b4_ref, out_ref):
    h = jnp.maximum(jnp.dot(x_ref[...], w1_ref[...], preferred_element_type=jnp.float32) + b1_ref[...], 0.0)
    h = jnp.maximum(jnp.dot(h, w2_ref[...], preferred_element_type=jnp.float32) + b2_ref[...], 0.0)
    h = jnp.maximum(jnp.dot(h, w3_ref[...], preferred_element_type=jnp.float32) + b3_ref[...], 0.0)
    y = jnp.dot(h, w4_ref[...], preferred_element_type=jnp.float32) + b4_ref[...]
    out_ref[...] = jax.nn.sigmoid(y)


@jax.jit
def kernel(demo, times, values, measurements, lengths, timescales,
           demo_w, phi_w, psi_w, W_k, W_q, rho_w):
    del psi_w

    a_proj = jnp.einsum('khd,hd->kh', W_k.reshape(-1, HEADS, DOT), W_q)[:D_IN]
    a_proj = a_proj / np.sqrt(float(DOT))
    (w1, b1) = phi_w[0]
    waug = jnp.zeros((D_IN, AUG), jnp.float32)
    waug = waug.at[:, :PHI_W].set(w1).at[:, PHI_W:PHI_W + HEADS].set(a_proj)
    baug = jnp.zeros((1, AUG), jnp.float32).at[:, :PHI_W].set(b1)
    wpv = waug[:N_POS + 1]
    wm = waug[N_POS + 1:]
    recip = 1.0 / timescales
    inv11 = jnp.concatenate([recip, recip, jnp.ones((1,), jnp.float32)])
    off11 = jnp.concatenate([jnp.zeros((NPH,), jnp.float32),
                             jnp.full((NPH,), np.pi / 2, jnp.float32),
                             jnp.zeros((1,), jnp.float32)])
    scw = jnp.stack([inv11, off11], axis=1)

    t2 = times.reshape(B, 1, T)
    v2 = values.reshape(B, 1, T)
    m2 = measurements.transpose(2, 0, 1).reshape(N_MOD, B, 1, T)
    d2 = demo.reshape(B, 1, D_DEMO)
    (wd1, bd1), (wd2, bd2) = demo_w
    flat_phi234 = []
    for w, bb in phi_w[1:]:
        flat_phi234 += [w.astype(jnp.bfloat16), bb.reshape(1, -1)]

    rep = lambda s: pl.BlockSpec(s, lambda b, L: tuple(0 for _ in s))
    w_specs = [rep((N_POS + 1, 2)),
               rep(wd1.shape), rep((1, PHI_W)), rep(wd2.shape), rep((1, D_IN)),
               rep(wpv.shape), rep(wm.shape), rep(waug.shape),
               rep((1, AUG))]
    for w, bb in phi_w[1:]:
        w_specs += [rep(w.shape), rep((1, w.shape[1]))]

    grid_spec = pltpu.PrefetchScalarGridSpec(
        num_scalar_prefetch=1,
        grid=(B,),
        in_specs=[
            pl.BlockSpec((1, 1, T), lambda b, L: (b, 0, 0)),
            pl.BlockSpec((1, 1, T), lambda b, L: (b, 0, 0)),
            pl.BlockSpec((N_MOD, 1, 1, T), lambda b, L: (0, b, 0, 0)),
            pl.BlockSpec((1, 1, D_DEMO), lambda b, L: (b, 0, 0)),
        ] + w_specs,
        out_specs=pl.BlockSpec((1, 1, HEADS * LATENT), lambda b, L: (b, 0, 0)),
        scratch_shapes=[pltpu.VMEM((TP, PHI_W), jnp.bfloat16)],
    )

    aggregated = pl.pallas_call(
        _main_body,
        grid_spec=grid_spec,
        out_shape=jax.ShapeDtypeStruct((B, 1, HEADS * LATENT), jnp.float32),
        compiler_params=pltpu.CompilerParams(
            dimension_semantics=("parallel",)),
    )(lengths, t2, v2, m2, d2,
      scw, wd1, bd1.reshape(1, -1), wd2, bd2.reshape(1, -1),
      wpv, wm, waug, baug, *flat_phi234)
    aggregated = aggregated.reshape(B, HEADS * LATENT)

    flat_rho = []
    for w, bb in rho_w:
        flat_rho += [w, bb.reshape(1, -1)]
    out = pl.pallas_call(
        _rho_body,
        out_shape=jax.ShapeDtypeStruct((B, 1), jnp.float32),
    )(aggregated, *flat_rho)
    return out
